# baseline (speedup 1.0000x reference)
import functools

import jax
import jax.numpy as jnp
from jax import lax
from jax.experimental import pallas as pl
from jax.experimental.pallas import tpu as pltpu
from jax.experimental.pallas import tpu_sc as plsc

T_STEPS = 16
L = 16


def _matvec_softmax_body(w_ref, state_ref, p_ref, acc_ref):
    ns = pl.program_id(0)

    @pl.when(ns == 0)
    def _():
        acc_ref[...] = jnp.zeros_like(acc_ref)

    w = w_ref[0, :].astype(jnp.bfloat16).astype(jnp.float32)
    blk = state_ref[...].astype(jnp.bfloat16).astype(jnp.float32)
    acc_ref[...] += jnp.sum(blk * w[None, :, None], axis=1)

    @pl.when(ns == pl.num_programs(0) - 1)
    def _():
        s = acc_ref[...]
        m = jnp.max(s, axis=-1, keepdims=True)
        e = jnp.exp(s - m)
        p_ref[...] = e / jnp.sum(e, axis=-1, keepdims=True)


def _tc_matvec_softmax(state, weight_matrix, half, bh):
    B, S, D = state.shape
    chunk = 128
    return pl.pallas_call(
        _matvec_softmax_body,
        grid=(S // chunk,),
        in_specs=[
            pl.BlockSpec((1, chunk), lambda i: (0, i)),
            pl.BlockSpec((bh, chunk, D), lambda i: (half, i, 0)),
        ],
        out_specs=pl.BlockSpec((bh, D), lambda i: (0, 0)),
        out_shape=jax.ShapeDtypeStruct((bh, D), jnp.float32),
        scratch_shapes=[pltpu.VMEM((bh, D), jnp.float32)],
        compiler_params=pltpu.CompilerParams(
            dimension_semantics=("arbitrary",),
        ),
    )(weight_matrix, state)


def _sc_mask_renorm(p):
    B, D = p.shape
    nchunks = D // L
    mesh = plsc.VectorSubcoreMesh(core_axis_name="c", subcore_axis_name="s")

    @functools.partial(
        pl.kernel,
        mesh=mesh,
        out_type=jax.ShapeDtypeStruct((B, T_STEPS, D), jnp.float32),
        scratch_types=[
            pltpu.VMEM((D,), jnp.float32),
            pltpu.VMEM((T_STEPS, D), jnp.float32),
            pltpu.VMEM((2 * L,), jnp.float32),
            pltpu.VMEM((2 * L,), jnp.int32),
        ],
    )
    def _sc_body(p_hbm, out_hbm, w_v, out_v, redf_v, redi_v):
        b = lax.axis_index("s") * 2 + lax.axis_index("c")

        @pl.when(b < B)
        def _():
            pltpu.sync_copy(p_hbm.at[b], w_v)
            lanes = lax.iota(jnp.int32, L)

            def _bfly_f(v, op):
                for k in (8, 4, 2, 1):
                    redf_v[pl.ds(0, L)] = v
                    redf_v[pl.ds(L, L)] = v
                    v = op(v, redf_v[pl.ds(k, L)])
                return v

            def _bfly_i_min(v):
                for k in (8, 4, 2, 1):
                    redi_v[pl.ds(0, L)] = v
                    redi_v[pl.ds(L, L)] = v
                    v = jnp.minimum(v, redi_v[pl.ds(k, L)])
                return v

            def _init(i, carry):
                svec, mvec, amvec = carry
                c = w_v[pl.ds(i * L, L)]
                out_v[0, pl.ds(i * L, L)] = c
                amvec = jnp.where(c > mvec, i, amvec)
                return svec + c, jnp.maximum(mvec, c), amvec

            svec, mvec, amvec = lax.fori_loop(
                0,
                nchunks,
                _init,
                (
                    jnp.zeros((L,), jnp.float32),
                    jnp.zeros((L,), jnp.float32),
                    jnp.zeros((L,), jnp.int32),
                ),
            )
            den = _bfly_f(svec, jnp.add)

            for t in range(1, T_STEPS):
                gmax = _bfly_f(mvec, jnp.maximum)
                pos = _bfly_i_min(
                    jnp.where(mvec == gmax, amvec * L + lanes, D)
                )
                den = den - gmax
                inv = jnp.float32(1.0) / den

                def _emit(i, carry):
                    mv, am = carry
                    c = w_v[pl.ds(i * L, L)]
                    c = jnp.where(i * L + lanes == pos, 0.0, c)
                    w_v[pl.ds(i * L, L)] = c
                    out_v[t, pl.ds(i * L, L)] = c * inv
                    am = jnp.where(c > mv, i, am)
                    return jnp.maximum(mv, c), am

                mvec, amvec = lax.fori_loop(
                    0,
                    nchunks,
                    _emit,
                    (jnp.zeros((L,), jnp.float32), jnp.zeros((L,), jnp.int32)),
                )

            pltpu.sync_copy(out_v, out_hbm.at[b])

    return _sc_body(p)


@jax.jit
def kernel(state, weight_matrix):
    B = state.shape[0]
    bh = B // 2
    p0 = _tc_matvec_softmax(state, weight_matrix, 0, bh)
    p1 = _tc_matvec_softmax(state, weight_matrix, 1, bh)
    out0 = _sc_mask_renorm(p0)
    out1 = _sc_mask_renorm(p1)
    return jnp.concatenate([out0, out1], axis=0)


# speedup vs baseline: 1.0857x; 1.0857x over previous
import functools

import jax
import jax.numpy as jnp
from jax import lax
from jax.experimental import pallas as pl
from jax.experimental.pallas import tpu as pltpu
from jax.experimental.pallas import tpu_sc as plsc

T_STEPS = 16
L = 16


def _matvec_softmax_body(w_ref, state_ref, p_ref, acc_ref):
    ns = pl.program_id(0)

    @pl.when(ns == 0)
    def _():
        acc_ref[...] = jnp.zeros_like(acc_ref)

    w = w_ref[0, :].astype(jnp.bfloat16).astype(jnp.float32)
    blk = state_ref[...].astype(jnp.bfloat16).astype(jnp.float32)
    acc_ref[...] += jnp.sum(blk * w[None, :, None], axis=1)

    @pl.when(ns == pl.num_programs(0) - 1)
    def _():
        s = acc_ref[...]
        m = jnp.max(s, axis=-1, keepdims=True)
        e = jnp.exp(s - m)
        p_ref[...] = e / jnp.sum(e, axis=-1, keepdims=True)


def _tc_matvec_softmax(state, weight_matrix):
    B, S, D = state.shape
    chunk = 128
    return pl.pallas_call(
        _matvec_softmax_body,
        grid=(S // chunk,),
        in_specs=[
            pl.BlockSpec((1, chunk), lambda i: (0, i)),
            pl.BlockSpec((B, chunk, D), lambda i: (0, i, 0)),
        ],
        out_specs=pl.BlockSpec((B, D), lambda i: (0, 0)),
        out_shape=jax.ShapeDtypeStruct((B, D), jnp.float32),
        scratch_shapes=[pltpu.VMEM((B, D), jnp.float32)],
        compiler_params=pltpu.CompilerParams(
            dimension_semantics=("arbitrary",),
        ),
    )(weight_matrix, state)


def _sc_mask_renorm(p):
    B, D = p.shape
    nchunks = D // L
    mesh = plsc.VectorSubcoreMesh(core_axis_name="c", subcore_axis_name="s")

    @functools.partial(
        pl.kernel,
        mesh=mesh,
        out_type=jax.ShapeDtypeStruct((B, T_STEPS, D), jnp.float32),
        scratch_types=[
            pltpu.VMEM((D,), jnp.float32),
            pltpu.VMEM((T_STEPS, D), jnp.float32),
            pltpu.VMEM((2 * L,), jnp.float32),
            pltpu.VMEM((2 * L,), jnp.int32),
        ],
    )
    def _sc_body(p_hbm, out_hbm, w_v, out_v, redf_v, redi_v):
        b = lax.axis_index("s") * 2 + lax.axis_index("c")

        @pl.when(b < B)
        def _():
            pltpu.sync_copy(p_hbm.at[b], w_v)
            pltpu.sync_copy(p_hbm.at[b], out_v.at[0])
            lanes = lax.iota(jnp.int32, L)

            def _bfly_f(v, op):
                for k in (8, 4, 2, 1):
                    redf_v[pl.ds(0, L)] = v
                    redf_v[pl.ds(L, L)] = v
                    v = op(v, redf_v[pl.ds(k, L)])
                return v

            def _bfly_i_min(v):
                for k in (8, 4, 2, 1):
                    redi_v[pl.ds(0, L)] = v
                    redi_v[pl.ds(L, L)] = v
                    v = jnp.minimum(v, redi_v[pl.ds(k, L)])
                return v

            def _init(i, carry):
                svec, mvec, amvec = carry
                c = w_v[pl.ds(i * L, L)]
                amvec = jnp.where(c > mvec, i, amvec)
                return svec + c, jnp.maximum(mvec, c), amvec

            svec, mvec, amvec = lax.fori_loop(
                0,
                nchunks,
                _init,
                (
                    jnp.zeros((L,), jnp.float32),
                    jnp.zeros((L,), jnp.float32),
                    jnp.zeros((L,), jnp.int32),
                ),
                unroll=4,
            )
            den = _bfly_f(svec, jnp.add)

            for t in range(1, T_STEPS):
                gmax = _bfly_f(mvec, jnp.maximum)
                pos = _bfly_i_min(
                    jnp.where(mvec == gmax, amvec * L + lanes, D)
                )
                den = den - gmax
                inv = jnp.float32(1.0) / den

                def _emit(i, carry):
                    mv, am = carry
                    c = w_v[pl.ds(i * L, L)]
                    c = jnp.where(i * L + lanes == pos, 0.0, c)
                    w_v[pl.ds(i * L, L)] = c
                    out_v[t, pl.ds(i * L, L)] = c * inv
                    am = jnp.where(c > mv, i, am)
                    return jnp.maximum(mv, c), am

                mvec, amvec = lax.fori_loop(
                    0,
                    nchunks,
                    _emit,
                    (jnp.zeros((L,), jnp.float32), jnp.zeros((L,), jnp.int32)),
                    unroll=4,
                )

            pltpu.sync_copy(out_v, out_hbm.at[b])

    return _sc_body(p)


@jax.jit
def kernel(state, weight_matrix):
    p = _tc_matvec_softmax(state, weight_matrix)
    return _sc_mask_renorm(p)


# speedup vs baseline: 1.1033x; 1.0162x over previous
import functools

import jax
import jax.numpy as jnp
from jax import lax
from jax.experimental import pallas as pl
from jax.experimental.pallas import tpu as pltpu
from jax.experimental.pallas import tpu_sc as plsc

T_STEPS = 16
L = 16


def _matvec_softmax_body(w_ref, state_ref, p_ref, acc_ref):
    ns = pl.program_id(0)

    @pl.when(ns == 0)
    def _():
        acc_ref[...] = jnp.zeros_like(acc_ref)

    w = w_ref[0, :].astype(jnp.bfloat16).astype(jnp.float32)
    blk = state_ref[...].astype(jnp.bfloat16).astype(jnp.float32)
    acc_ref[...] += jnp.sum(blk * w[None, :, None], axis=1)

    @pl.when(ns == pl.num_programs(0) - 1)
    def _():
        s = acc_ref[...]
        m = jnp.max(s, axis=-1, keepdims=True)
        e = jnp.exp(s - m)
        p_ref[...] = e / jnp.sum(e, axis=-1, keepdims=True)


def _tc_matvec_softmax(state, weight_matrix):
    B, S, D = state.shape
    chunk = 128
    return pl.pallas_call(
        _matvec_softmax_body,
        grid=(S // chunk,),
        in_specs=[
            pl.BlockSpec((1, chunk), lambda i: (0, i)),
            pl.BlockSpec((B, chunk, D), lambda i: (0, i, 0)),
        ],
        out_specs=pl.BlockSpec((B, D), lambda i: (0, 0)),
        out_shape=jax.ShapeDtypeStruct((B, D), jnp.float32),
        scratch_shapes=[pltpu.VMEM((B, D), jnp.float32)],
        compiler_params=pltpu.CompilerParams(
            dimension_semantics=("arbitrary",),
        ),
    )(weight_matrix, state)


def _sc_mask_renorm(p):
    B, D = p.shape
    nchunks = D // L
    mesh = plsc.VectorSubcoreMesh(core_axis_name="c", subcore_axis_name="s")

    @functools.partial(
        pl.kernel,
        mesh=mesh,
        out_type=jax.ShapeDtypeStruct((B, T_STEPS, D), jnp.float32),
        scratch_types=[
            pltpu.VMEM((D,), jnp.float32),
            pltpu.VMEM((T_STEPS, D), jnp.float32),
            pltpu.VMEM((2 * L,), jnp.float32),
            pltpu.VMEM((2 * L,), jnp.int32),
        ],
    )
    def _sc_body(p_hbm, out_hbm, w_v, out_v, redf_v, redi_v):
        b = lax.axis_index("s") * 2 + lax.axis_index("c")

        @pl.when(b < B)
        def _():
            pltpu.sync_copy(p_hbm.at[b], w_v)
            pltpu.sync_copy(p_hbm.at[b], out_v.at[0])
            lanes = lax.iota(jnp.int32, L)

            def _bfly_f(v, op):
                for k in (8, 4, 2, 1):
                    redf_v[pl.ds(0, L)] = v
                    redf_v[pl.ds(L, L)] = v
                    v = op(v, redf_v[pl.ds(k, L)])
                return v

            def _bfly_i_min(v):
                for k in (8, 4, 2, 1):
                    redi_v[pl.ds(0, L)] = v
                    redi_v[pl.ds(L, L)] = v
                    v = jnp.minimum(v, redi_v[pl.ds(k, L)])
                return v

            def _init(i, carry):
                svec, mvec, amvec = carry
                c = w_v[pl.ds(i * L, L)]
                amvec = jnp.where(c > mvec, i, amvec)
                return svec + c, jnp.maximum(mvec, c), amvec

            svec, mvec, amvec = lax.fori_loop(
                0,
                nchunks,
                _init,
                (
                    jnp.zeros((L,), jnp.float32),
                    jnp.zeros((L,), jnp.float32),
                    jnp.zeros((L,), jnp.int32),
                ),
                unroll=4,
            )
            den = _bfly_f(svec, jnp.add)

            def _step(t, carry):
                den, mvec, amvec = carry
                gmax = _bfly_f(mvec, jnp.maximum)
                pos = _bfly_i_min(
                    jnp.where(mvec == gmax, amvec * L + lanes, D)
                )
                den = den - gmax
                inv = jnp.float32(1.0) / den

                def _emit(i, carry2):
                    mv, am = carry2
                    c = w_v[pl.ds(i * L, L)]
                    c = jnp.where(i * L + lanes == pos, 0.0, c)
                    w_v[pl.ds(i * L, L)] = c
                    out_v[t, pl.ds(i * L, L)] = c * inv
                    am = jnp.where(c > mv, i, am)
                    return jnp.maximum(mv, c), am

                mvec, amvec = lax.fori_loop(
                    0,
                    nchunks,
                    _emit,
                    (jnp.zeros((L,), jnp.float32), jnp.zeros((L,), jnp.int32)),
                    unroll=4,
                )
                return den, mvec, amvec

            lax.fori_loop(1, T_STEPS, _step, (den, mvec, amvec))

            pltpu.sync_copy(out_v, out_hbm.at[b])

    return _sc_body(p)


@jax.jit
def kernel(state, weight_matrix):
    p = _tc_matvec_softmax(state, weight_matrix)
    return _sc_mask_renorm(p)
